# Initial kernel scaffold; baseline (speedup 1.0000x reference)
import jax
import jax.numpy as jnp
from jax.experimental import pallas as pl


def kernel(flat_obs, segment_ids, W1, ln1_g, ln1_b, W2, ln2_g, ln2_b, W_ih, W_hh, b):
    raise NotImplementedError("write your pallas kernel here")


# speedup vs baseline: 1.5390x; 1.5390x over previous
import functools

import jax
import jax.numpy as jnp
from jax import lax
from jax.experimental import pallas as pl
from jax.experimental.pallas import tpu as pltpu
from jax.experimental.pallas import tpu_sc as plsc

IN_FEAT = 128
FEATURES = 512
HIDDEN = 512
N_TIME = 32
N_PLAYERS = 16
TOTAL = 32768
NUM_SEG = N_TIME * N_PLAYERS


_BLK = 2048


def _embed_body(x_ref, w1_ref, g1_ref, b1_ref, w2_ref, g2_ref, b2_ref, o_ref):
    x = x_ref[...]
    h = jnp.dot(x, w1_ref[...], preferred_element_type=jnp.float32)
    h = jnp.where(h >= 0, h, 0.1 * h)
    mu = jnp.mean(h, axis=-1, keepdims=True)
    var = jnp.mean((h - mu) ** 2, axis=-1, keepdims=True)
    h = (h - mu) / jnp.sqrt(var + 1e-5) * g1_ref[...] + b1_ref[...]
    e = jnp.dot(h, w2_ref[...], preferred_element_type=jnp.float32)
    e = jnp.where(e >= 0, e, 0.1 * e)
    mu2 = jnp.mean(e, axis=-1, keepdims=True)
    var2 = jnp.mean((e - mu2) ** 2, axis=-1, keepdims=True)
    o_ref[...] = (e - mu2) / jnp.sqrt(var2 + 1e-5) * g2_ref[...] + b2_ref[...]


def _embed(flat_obs, W1, g1, b1, W2, g2, b2):
    grid = (TOTAL // _BLK,)
    return pl.pallas_call(
        _embed_body,
        grid=grid,
        in_specs=[
            pl.BlockSpec((_BLK, IN_FEAT), lambda i: (i, 0)),
            pl.BlockSpec((IN_FEAT, FEATURES // 2), lambda i: (0, 0)),
            pl.BlockSpec((1, FEATURES // 2), lambda i: (0, 0)),
            pl.BlockSpec((1, FEATURES // 2), lambda i: (0, 0)),
            pl.BlockSpec((FEATURES // 2, FEATURES), lambda i: (0, 0)),
            pl.BlockSpec((1, FEATURES), lambda i: (0, 0)),
            pl.BlockSpec((1, FEATURES), lambda i: (0, 0)),
        ],
        out_specs=pl.BlockSpec((_BLK, FEATURES), lambda i: (i, 0)),
        out_shape=jax.ShapeDtypeStruct((TOTAL, FEATURES), jnp.float32),
    )(flat_obs, W1, g1, b1, W2, g2, b2)


_NC = 2
_NS = 16
_NW = _NC * _NS
_NCG = 4
_NRG = _NW // _NCG
_CW = FEATURES // _NCG
_RPW = TOTAL // _NRG
_CHUNK = 128


def _segsum_body(emb_hbm, ids_hbm, out_hbm, idx_v, rows_v, acc_v):
    c = lax.axis_index("c")
    s = lax.axis_index("s")
    wid = s * _NC + c
    cg = wid % _NCG
    rg = wid // _NCG
    col0 = cg * _CW
    zero16 = jnp.zeros((16,), jnp.float32)

    def zbody(i, carry):
        for j in range(_CW // 16):
            acc_v[i, pl.ds(j * 16, 16)] = zero16
        return carry

    lax.fori_loop(0, NUM_SEG, zbody, 0)

    def chunk_body(k, carry):
        base = rg * _RPW + k * _CHUNK
        pltpu.sync_copy(ids_hbm.at[pl.ds(base, _CHUNK)], idx_v)
        pltpu.sync_copy(emb_hbm.at[pl.ds(base, _CHUNK), pl.ds(col0, _CW)], rows_v)

        def grp_body(g, carry2):
            ids16 = idx_v[pl.ds(g * 16, 16)]
            r0 = g * 16
            for rr in range(16):
                seg = ids16[rr]
                for j in range(_CW // 16):
                    plsc.addupdate(
                        acc_v.at[seg, pl.ds(j * 16, 16)],
                        rows_v[r0 + rr, pl.ds(j * 16, 16)],
                    )
            return carry2

        lax.fori_loop(0, _CHUNK // 16, grp_body, 0)
        return carry

    lax.fori_loop(0, _RPW // _CHUNK, chunk_body, 0)
    pltpu.sync_copy(acc_v, out_hbm.at[rg, :, pl.ds(col0, _CW)])


def _segsum(emb, ids):
    mesh = plsc.VectorSubcoreMesh(core_axis_name="c", subcore_axis_name="s")
    return pl.kernel(
        _segsum_body,
        mesh=mesh,
        out_type=jax.ShapeDtypeStruct((_NRG, NUM_SEG, FEATURES), jnp.float32),
        scratch_types=[
            pltpu.VMEM((_CHUNK,), jnp.int32),
            pltpu.VMEM((_CHUNK, _CW), jnp.float32),
            pltpu.VMEM((NUM_SEG, _CW), jnp.float32),
        ],
    )(emb, ids)


def _lstm_body(p_ref, wih_ref, whh_ref, b_ref, o_ref, xw_s, h_s, c_s):
    t = pl.program_id(0)

    @pl.when(t == 0)
    def _():
        seq = jnp.sum(p_ref[...], axis=0)
        xw_s[...] = jnp.dot(
            seq,
            wih_ref[...],
            preferred_element_type=jnp.float32,
            precision=jax.lax.Precision.HIGHEST,
        )
        h_s[...] = jnp.zeros((N_PLAYERS, HIDDEN), jnp.float32)
        c_s[...] = jnp.zeros((N_PLAYERS, HIDDEN), jnp.float32)

    h = h_s[...]
    gates = (
        xw_s[pl.ds(t * N_PLAYERS, N_PLAYERS), :]
        + jnp.dot(
            h,
            whh_ref[...],
            preferred_element_type=jnp.float32,
            precision=jax.lax.Precision.HIGHEST,
        )
    ) + b_ref[...]
    i = jax.nn.sigmoid(gates[:, :HIDDEN])
    f = jax.nn.sigmoid(gates[:, HIDDEN : 2 * HIDDEN])
    g = jnp.tanh(gates[:, 2 * HIDDEN : 3 * HIDDEN])
    o = jax.nn.sigmoid(gates[:, 3 * HIDDEN :])
    c_new = f * c_s[...] + i * g
    h_new = o * jnp.tanh(c_new)
    h_s[...] = h_new
    c_s[...] = c_new
    o_ref[...] = h_new[None]


def _lstm(pooled, W_ih, W_hh, b):
    return pl.pallas_call(
        _lstm_body,
        grid=(N_TIME,),
        in_specs=[
            pl.BlockSpec((_NRG, NUM_SEG, FEATURES), lambda t: (0, 0, 0)),
            pl.BlockSpec((FEATURES, 4 * HIDDEN), lambda t: (0, 0)),
            pl.BlockSpec((HIDDEN, 4 * HIDDEN), lambda t: (0, 0)),
            pl.BlockSpec((1, 4 * HIDDEN), lambda t: (0, 0)),
        ],
        out_specs=pl.BlockSpec((1, N_PLAYERS, HIDDEN), lambda t: (t, 0, 0)),
        out_shape=jax.ShapeDtypeStruct((N_TIME, N_PLAYERS, HIDDEN), jnp.float32),
        scratch_shapes=[
            pltpu.VMEM((NUM_SEG, 4 * HIDDEN), jnp.float32),
            pltpu.VMEM((N_PLAYERS, HIDDEN), jnp.float32),
            pltpu.VMEM((N_PLAYERS, HIDDEN), jnp.float32),
        ],
    )(pooled, W_ih, W_hh, b)


def kernel(flat_obs, segment_ids, W1, ln1_g, ln1_b, W2, ln2_g, ln2_b, W_ih, W_hh, b):
    ids = segment_ids.astype(jnp.int32)
    emb = _embed(
        flat_obs,
        W1,
        ln1_g.reshape(1, -1),
        ln1_b.reshape(1, -1),
        W2,
        ln2_g.reshape(1, -1),
        ln2_b.reshape(1, -1),
    )
    pooled = _segsum(emb, ids)
    return _lstm(pooled, W_ih, W_hh, b.reshape(1, -1))


# speedup vs baseline: 1.8648x; 1.2117x over previous
import functools

import jax
import jax.numpy as jnp
from jax import lax
from jax.experimental import pallas as pl
from jax.experimental.pallas import tpu as pltpu
from jax.experimental.pallas import tpu_sc as plsc

IN_FEAT = 128
FEATURES = 512
HIDDEN = 512
N_TIME = 32
N_PLAYERS = 16
TOTAL = 32768
NUM_SEG = N_TIME * N_PLAYERS


_BLK = 2048


def _embed_body(x_ref, w1_ref, g1_ref, b1_ref, w2_ref, g2_ref, b2_ref, o_ref):
    x = x_ref[...]
    h = jnp.dot(x, w1_ref[...], preferred_element_type=jnp.float32)
    h = jnp.where(h >= 0, h, 0.1 * h)
    mu = jnp.mean(h, axis=-1, keepdims=True)
    var = jnp.mean((h - mu) ** 2, axis=-1, keepdims=True)
    h = (h - mu) / jnp.sqrt(var + 1e-5) * g1_ref[...] + b1_ref[...]
    e = jnp.dot(h, w2_ref[...], preferred_element_type=jnp.float32)
    e = jnp.where(e >= 0, e, 0.1 * e)
    mu2 = jnp.mean(e, axis=-1, keepdims=True)
    var2 = jnp.mean((e - mu2) ** 2, axis=-1, keepdims=True)
    o_ref[...] = (e - mu2) / jnp.sqrt(var2 + 1e-5) * g2_ref[...] + b2_ref[...]


def _embed(flat_obs, W1, g1, b1, W2, g2, b2):
    grid = (TOTAL // _BLK,)
    return pl.pallas_call(
        _embed_body,
        grid=grid,
        in_specs=[
            pl.BlockSpec((_BLK, IN_FEAT), lambda i: (i, 0)),
            pl.BlockSpec((IN_FEAT, FEATURES // 2), lambda i: (0, 0)),
            pl.BlockSpec((1, FEATURES // 2), lambda i: (0, 0)),
            pl.BlockSpec((1, FEATURES // 2), lambda i: (0, 0)),
            pl.BlockSpec((FEATURES // 2, FEATURES), lambda i: (0, 0)),
            pl.BlockSpec((1, FEATURES), lambda i: (0, 0)),
            pl.BlockSpec((1, FEATURES), lambda i: (0, 0)),
        ],
        out_specs=pl.BlockSpec((_BLK, FEATURES), lambda i: (i, 0)),
        out_shape=jax.ShapeDtypeStruct((TOTAL, FEATURES), jnp.float32),
    )(flat_obs, W1, g1, b1, W2, g2, b2)


_NC = 2
_NS = 16
_NW = _NC * _NS
_NCG = 4
_NRG = _NW // _NCG
_CW = FEATURES // _NCG
_RPW = TOTAL // _NRG
_CHUNK = 128


def _segsum_body(
    emb_hbm, ids_hbm, out_hbm,
    idx0, idx1, rows0, rows1, acc_v,
    semi0, semi1, semd0, semd1,
):
    c = lax.axis_index("c")
    s = lax.axis_index("s")
    wid = s * _NC + c
    cg = wid % _NCG
    rg = wid // _NCG
    col0 = cg * _CW
    zero16 = jnp.zeros((16,), jnp.float32)
    idx = (idx0, idx1)
    rows = (rows0, rows1)
    semi = (semi0, semi1)
    semd = (semd0, semd1)
    nch = _RPW // _CHUNK

    def start(k, b):
        base = rg * _RPW + k * _CHUNK
        pltpu.make_async_copy(ids_hbm.at[pl.ds(base, _CHUNK)], idx[b], semi[b]).start()
        pltpu.make_async_copy(
            emb_hbm.at[pl.ds(base, _CHUNK), pl.ds(col0, _CW)], rows[b], semd[b]
        ).start()

    def wait(k, b):
        base = rg * _RPW + k * _CHUNK
        pltpu.make_async_copy(ids_hbm.at[pl.ds(base, _CHUNK)], idx[b], semi[b]).wait()
        pltpu.make_async_copy(
            emb_hbm.at[pl.ds(base, _CHUNK), pl.ds(col0, _CW)], rows[b], semd[b]
        ).wait()

    start(0, 0)

    def zbody(i, carry):
        for j in range(_CW // 16):
            acc_v[i, pl.ds(j * 16, 16)] = zero16
        return carry

    lax.fori_loop(0, NUM_SEG, zbody, 0)

    def accumulate(idx_v, rows_v):
        def grp_body(g, carry2):
            ids16 = idx_v[pl.ds(g * 16, 16)]
            r0 = g * 16
            segs = [ids16[rr] for rr in range(16)]
            for rr in range(16):
                seg = segs[rr]
                for j in range(_CW // 16):
                    plsc.addupdate(
                        acc_v.at[seg, pl.ds(j * 16, 16)],
                        rows_v[r0 + rr, pl.ds(j * 16, 16)],
                    )
            return carry2

        lax.fori_loop(0, _CHUNK // 16, grp_body, 0)

    def chunk_body(k2, carry):
        for b in range(2):
            k = k2 * 2 + b

            @pl.when(k + 1 < nch)
            def _():
                start(k + 1, 1 - b)

            wait(k, b)
            accumulate(idx[b], rows[b])
        return carry

    lax.fori_loop(0, nch // 2, chunk_body, 0)
    pltpu.sync_copy(acc_v, out_hbm.at[rg, :, pl.ds(col0, _CW)])


def _segsum(emb, ids):
    mesh = plsc.VectorSubcoreMesh(core_axis_name="c", subcore_axis_name="s")
    return pl.kernel(
        _segsum_body,
        mesh=mesh,
        out_type=jax.ShapeDtypeStruct((_NRG, NUM_SEG, FEATURES), jnp.float32),
        scratch_types=[
            pltpu.VMEM((_CHUNK,), jnp.int32),
            pltpu.VMEM((_CHUNK,), jnp.int32),
            pltpu.VMEM((_CHUNK, _CW), jnp.float32),
            pltpu.VMEM((_CHUNK, _CW), jnp.float32),
            pltpu.VMEM((NUM_SEG, _CW), jnp.float32),
            pltpu.SemaphoreType.DMA,
            pltpu.SemaphoreType.DMA,
            pltpu.SemaphoreType.DMA,
            pltpu.SemaphoreType.DMA,
        ],
    )(emb, ids)


def _lstm_body(p_ref, wih_ref, whh_ref, b_ref, o_ref, xw_s, h_s, c_s):
    t = pl.program_id(0)

    @pl.when(t == 0)
    def _():
        seq = jnp.sum(p_ref[...], axis=0)
        xw_s[...] = jnp.dot(
            seq,
            wih_ref[...],
            preferred_element_type=jnp.float32,
            precision=jax.lax.Precision.HIGHEST,
        )
        h_s[...] = jnp.zeros((N_PLAYERS, HIDDEN), jnp.float32)
        c_s[...] = jnp.zeros((N_PLAYERS, HIDDEN), jnp.float32)

    h = h_s[...]
    gates = (
        xw_s[pl.ds(t * N_PLAYERS, N_PLAYERS), :]
        + jnp.dot(
            h,
            whh_ref[...],
            preferred_element_type=jnp.float32,
            precision=jax.lax.Precision.HIGHEST,
        )
    ) + b_ref[...]
    i = jax.nn.sigmoid(gates[:, :HIDDEN])
    f = jax.nn.sigmoid(gates[:, HIDDEN : 2 * HIDDEN])
    g = jnp.tanh(gates[:, 2 * HIDDEN : 3 * HIDDEN])
    o = jax.nn.sigmoid(gates[:, 3 * HIDDEN :])
    c_new = f * c_s[...] + i * g
    h_new = o * jnp.tanh(c_new)
    h_s[...] = h_new
    c_s[...] = c_new
    o_ref[...] = h_new[None]


def _lstm(pooled, W_ih, W_hh, b):
    return pl.pallas_call(
        _lstm_body,
        grid=(N_TIME,),
        in_specs=[
            pl.BlockSpec((_NRG, NUM_SEG, FEATURES), lambda t: (0, 0, 0)),
            pl.BlockSpec((FEATURES, 4 * HIDDEN), lambda t: (0, 0)),
            pl.BlockSpec((HIDDEN, 4 * HIDDEN), lambda t: (0, 0)),
            pl.BlockSpec((1, 4 * HIDDEN), lambda t: (0, 0)),
        ],
        out_specs=pl.BlockSpec((1, N_PLAYERS, HIDDEN), lambda t: (t, 0, 0)),
        out_shape=jax.ShapeDtypeStruct((N_TIME, N_PLAYERS, HIDDEN), jnp.float32),
        scratch_shapes=[
            pltpu.VMEM((NUM_SEG, 4 * HIDDEN), jnp.float32),
            pltpu.VMEM((N_PLAYERS, HIDDEN), jnp.float32),
            pltpu.VMEM((N_PLAYERS, HIDDEN), jnp.float32),
        ],
    )(pooled, W_ih, W_hh, b)


def kernel(flat_obs, segment_ids, W1, ln1_g, ln1_b, W2, ln2_g, ln2_b, W_ih, W_hh, b):
    ids = segment_ids.astype(jnp.int32)
    emb = _embed(
        flat_obs,
        W1,
        ln1_g.reshape(1, -1),
        ln1_b.reshape(1, -1),
        W2,
        ln2_g.reshape(1, -1),
        ln2_b.reshape(1, -1),
    )
    pooled = _segsum(emb, ids)
    return _lstm(pooled, W_ih, W_hh, b.reshape(1, -1))
